# baseline (speedup 1.0000x reference)
import jax
import jax.numpy as jnp
from jax import lax
from jax.experimental import pallas as pl
from jax.experimental.pallas import tpu as pltpu
from jax.experimental.pallas import tpu_sc as plsc

N_NODES = 50000
N_PAD = 50048
N_USERS = 25000
N_EDGES = 800000
DIM = 64
HALF = 32
XDIM = 16
BATCH = 4096
LAYERS = 3

NC = 2
NS = 16
LANES = 16

CK = 128
CHUNKS_PER_TILE = 392
EDGES_PER_TILE = CHUNKS_PER_TILE * CK
E_PAD = EDGES_PER_TILE * NS
SUPER = 14
CPS = CHUNKS_PER_TILE // SUPER
NBUF = 4
NMSG = 2

ROWS_PER_TILE = N_PAD // NS

BPT = BATCH // NS
BCH = BPT // CK


def _unpack_scale(gref, mref, val_ref, j):
  @plsc.parallel_loop(0, CK // LANES, unroll=2)
  def group(g):
    vals = val_ref[j, pl.ds(g * LANES, LANES)]
    for l in range(LANES):
      idx = jnp.full((LANES,), l, dtype=jnp.int32)
      vv = vals.at[idx].get(mode='promise_in_bounds')
      e = g * LANES + l
      xb = gref[e, :]
      a, b = plsc.unpack(xb, format=plsc.PackFormat.INTERLEAVED)
      mref[e, pl.ds(0, LANES)] = a.astype(jnp.float32) * vv
      mref[e, pl.ds(LANES, LANES)] = b.astype(jnp.float32) * vv


def _layer_body(zeros_hbm, cb0, cb1, col2d, row2d, val2d, nxt0, nxt1,
                acc, colv, rowv, valv, gb0, gb1, gb2, gb3, m0, m1,
                g0, g1, g2, g3, s0, s1):
  gbufs = (gb0, gb1, gb2, gb3)
  mbufs = (m0, m1)
  gsems = (g0, g1, g2, g3)
  ssems = (s0, s1)
  c = lax.axis_index("c")
  s = lax.axis_index("s")
  zsl = pl.ds(s * ROWS_PER_TILE, ROWS_PER_TILE)

  pltpu.sync_copy(zeros_hbm.at[zsl], acc.at[zsl])
  plsc.subcore_barrier()

  base_chunk = s * CHUNKS_PER_TILE

  def fire_g(j, b, guard):
    @pl.when(jnp.logical_and(guard, c == 0))
    def _g0():
      pltpu.async_copy(cb0.at[colv.at[j]], gbufs[b], gsems[b])
    @pl.when(jnp.logical_and(guard, c == 1))
    def _g1():
      pltpu.async_copy(cb1.at[colv.at[j]], gbufs[b], gsems[b])

  def wait_g(b):
    pltpu.make_async_copy(cb0.at[pl.ds(0, CK)], gbufs[b], gsems[b]).wait()

  def fire_s(j, mb):
    pltpu.async_copy(mbufs[mb], acc.at[rowv.at[j]], ssems[mb], add=True)

  def wait_s(mb, guard=None):
    def _w():
      pltpu.make_async_copy(mbufs[mb], acc.at[pl.ds(0, CK)], ssems[mb]).wait()
    if guard is None:
      _w()
    else:
      pl.when(guard)(_w)

  def super_step(sc_i, _):
    pltpu.sync_copy(col2d.at[pl.ds(base_chunk + sc_i * CPS, CPS)], colv)
    pltpu.sync_copy(row2d.at[pl.ds(base_chunk + sc_i * CPS, CPS)], rowv)
    pltpu.sync_copy(val2d.at[pl.ds(base_chunk + sc_i * CPS, CPS)], valv)

    fire_g(0, 0, True)
    fire_g(1, 1, True)

    def quad(q, __):
      for u in range(NBUF):
        j = q * NBUF + u
        mb = u % NMSG
        wait_s(mb, guard=j >= 2)
        fire_g(j + 2, (u + 2) % NBUF, j + 2 <= CPS - 1)
        wait_g(u)
        _unpack_scale(gbufs[u], mbufs[mb], valv, j)
        fire_s(j, mb)
      return __
    lax.fori_loop(0, CPS // NBUF, quad, None)

    wait_s(0)
    wait_s(1)
    return _
  lax.fori_loop(0, SUPER, super_step, None)

  plsc.subcore_barrier()

  @pl.when(c == 0)
  def _w0():
    pltpu.sync_copy(acc.at[zsl], nxt0.at[zsl])
  @pl.when(c == 1)
  def _w1():
    pltpu.sync_copy(acc.at[zsl], nxt1.at[zsl])


def _make_layer_kernel():
  nt = jax.ShapeDtypeStruct((N_PAD, HALF), jnp.float32)
  return pl.kernel(
      _layer_body,
      out_type=(nt, nt),
      mesh=plsc.VectorSubcoreMesh(
          core_axis_name="c", subcore_axis_name="s",
          num_cores=NC, num_subcores=NS),
      compiler_params=pltpu.CompilerParams(
          use_tc_tiling_on_sc=False, needs_layout_passes=False),
      scratch_types=(
          [pltpu.VMEM_SHARED((N_PAD, HALF), jnp.float32)]
          + [pltpu.VMEM((CPS, CK), jnp.int32)] * 2
          + [pltpu.VMEM((CPS, CK), jnp.float32)]
          + [pltpu.VMEM((CK, HALF), jnp.bfloat16)] * NBUF
          + [pltpu.VMEM((CK, HALF), jnp.float32)] * NMSG
          + [pltpu.SemaphoreType.DMA] * (NBUF + NMSG)
      ),
  )


def _gather_sum_body(t0, t1, a0, a1, b0, b1, c0, c1, uix, iix,
                     usum0, usum1, isum0, isum1,
                     idxv, gbuf, sbuf, sem):
  c = lax.axis_index("c")
  s = lax.axis_index("s")
  pairs = ((t0, t1), (a0, a1), (b0, b1), (c0, c1))

  def one_batch(idx3d, out0, out1):
    pltpu.sync_copy(idx3d.at[s], idxv)
    for j in range(BCH):
      for ti, (tb0, tb1) in enumerate(pairs):
        @pl.when(c == 0)
        def _g0(tb0=tb0, j=j):
          pltpu.async_copy(tb0.at[idxv.at[j]], gbuf, sem).wait()
        @pl.when(c == 1)
        def _g1(tb1=tb1, j=j):
          pltpu.async_copy(tb1.at[idxv.at[j]], gbuf, sem).wait()

        def body(r, _, first=(ti == 0)):
          for d2 in range(HALF // LANES):
            dsl = pl.ds(d2 * LANES, LANES)
            if first:
              sbuf[r, dsl] = gbuf[r, dsl]
            else:
              sbuf[r, dsl] = sbuf[r, dsl] + gbuf[r, dsl]
          return _
        lax.fori_loop(0, CK, body, None)
      osl = pl.ds(s * BPT + j * CK, CK)
      @pl.when(c == 0)
      def _o0(osl=osl):
        pltpu.sync_copy(sbuf, out0.at[osl])
      @pl.when(c == 1)
      def _o1(osl=osl):
        pltpu.sync_copy(sbuf, out1.at[osl])

  one_batch(uix, usum0, usum1)
  one_batch(iix, isum0, isum1)


def _make_gather_sum_kernel():
  bt = jax.ShapeDtypeStruct((BATCH, HALF), jnp.float32)
  return pl.kernel(
      _gather_sum_body,
      out_type=(bt, bt, bt, bt),
      mesh=plsc.VectorSubcoreMesh(
          core_axis_name="c", subcore_axis_name="s",
          num_cores=NC, num_subcores=NS),
      compiler_params=pltpu.CompilerParams(
          use_tc_tiling_on_sc=False, needs_layout_passes=False),
      scratch_types=[
          pltpu.VMEM((BCH, CK), jnp.int32),
          pltpu.VMEM((CK, HALF), jnp.float32),
          pltpu.VMEM((CK, HALF), jnp.float32),
          pltpu.SemaphoreType.DMA,
      ],
  )


def _forward_body(us_ref, is_ref, xij_ref, xt_ref, out_ref):
  ue = us_ref[...] * 0.25
  ie = is_ref[...] * 0.25
  xe = xt_ref[...] * (xij_ref[...] - 0.3).reshape(-1, 1)
  u = jnp.concatenate([ue, xe], axis=1)
  v = jnp.concatenate([ie, xe], axis=1)
  u = jax.nn.sigmoid(u)
  v = v - jnp.max(v, axis=1, keepdims=True)
  ev = jnp.exp(v)
  v = ev / jnp.sum(ev, axis=1, keepdims=True)
  out_ref[...] = jnp.sum(u * v, axis=1, keepdims=True)


def _bf16_interleave(x):
  xb = x.astype(jnp.bfloat16)
  return xb.reshape(N_PAD, 2, LANES).transpose(0, 2, 1).reshape(N_PAD, HALF)


def kernel(users, items, xij, edge_index, edge_vals, user_table, item_table,
           xij_table):
  all_emb = jnp.concatenate([user_table, item_table], axis=0)
  all_emb = jnp.pad(all_emb, ((0, N_PAD - N_NODES), (0, 0)))
  cur0 = all_emb[:, :HALF]
  cur1 = all_emb[:, HALF:]

  pad = E_PAD - N_EDGES
  row = jnp.pad(edge_index[0], (0, pad)).reshape(E_PAD // CK, CK)
  col = jnp.pad(edge_index[1], (0, pad)).reshape(E_PAD // CK, CK)
  val = jnp.pad(edge_vals, (0, pad)).reshape(E_PAD // CK, CK)

  zeros = jnp.zeros((N_PAD, HALF), jnp.float32)
  layer = _make_layer_kernel()
  tabs = [(cur0, cur1)]
  for _ in range(LAYERS):
    p0, p1 = tabs[-1]
    tabs.append(layer(zeros, _bf16_interleave(p0), _bf16_interleave(p1),
                      col, row, val))

  uix = users.reshape(NS, BCH, CK)
  iix = (items + N_USERS).reshape(NS, BCH, CK)
  gsum = _make_gather_sum_kernel()
  usum0, usum1, isum0, isum1 = gsum(
      tabs[0][0], tabs[0][1], tabs[1][0], tabs[1][1],
      tabs[2][0], tabs[2][1], tabs[3][0], tabs[3][1], uix, iix)

  usum = jnp.concatenate([usum0, usum1], axis=1)
  isum = jnp.concatenate([isum0, isum1], axis=1)
  xt = jnp.tile(xij_table, (BATCH, 1))

  gamma = pl.pallas_call(
      _forward_body,
      out_shape=jax.ShapeDtypeStruct((BATCH, 1), jnp.float32),
  )(usum, isum, xij, xt)
  return gamma.reshape(BATCH)


# speedup vs baseline: 1.4871x; 1.4871x over previous
import jax
import jax.numpy as jnp
from jax import lax
from jax.experimental import pallas as pl
from jax.experimental.pallas import tpu as pltpu
from jax.experimental.pallas import tpu_sc as plsc

N_NODES = 50000
N_PAD = 50048
N_USERS = 25000
N_EDGES = 800000
DIM = 64
HALF = 32
XDIM = 16
BATCH = 4096
LAYERS = 3

NC = 2
NS = 16
LANES = 16

CK = 128
CHUNKS_PER_TILE = 392
EDGES_PER_TILE = CHUNKS_PER_TILE * CK
E_PAD = EDGES_PER_TILE * NS
SUPER = 14
CPS = CHUNKS_PER_TILE // SUPER
NBUF = 4

ROWS_PER_TILE = N_PAD // NS

BPT = BATCH // NS
BCH = BPT // CK


def _scale_rows(rows_v, val_ref, j):
  @plsc.parallel_loop(0, CK // LANES, unroll=2)
  def group(g):
    vals = val_ref[j, pl.ds(g * LANES, LANES)]
    for l in range(LANES):
      idx = jnp.full((LANES,), l, dtype=jnp.int32)
      vv = vals.at[idx].get(mode='promise_in_bounds')
      e = g * LANES + l
      for d2 in range(HALF // LANES):
        sl = pl.ds(d2 * LANES, LANES)
        rows_v[e, sl] = rows_v[e, sl] * vv


def _edge_pass(c, s, cur0, cur1, col2d, row2d, val2d,
               acc, colv, rowv, valv, rows, gsems, ssems):
  base_chunk = s * CHUNKS_PER_TILE

  def fire_g(j, b, guard):
    @pl.when(jnp.logical_and(guard, c == 0))
    def _g0():
      pltpu.async_copy(cur0.at[colv.at[j]], rows[b], gsems[b])
    @pl.when(jnp.logical_and(guard, c == 1))
    def _g1():
      pltpu.async_copy(cur1.at[colv.at[j]], rows[b], gsems[b])

  def wait_g(b):
    pltpu.make_async_copy(cur0.at[pl.ds(0, CK)], rows[b], gsems[b]).wait()

  def fire_s(j, b):
    pltpu.async_copy(rows[b], acc.at[rowv.at[j]], ssems[b], add=True)

  def wait_s(b, guard=None):
    def _w():
      pltpu.make_async_copy(rows[b], acc.at[pl.ds(0, CK)], ssems[b]).wait()
    if guard is None:
      _w()
    else:
      pl.when(guard)(_w)

  def super_step(sc_i, _):
    pltpu.sync_copy(col2d.at[pl.ds(base_chunk + sc_i * CPS, CPS)], colv)
    pltpu.sync_copy(row2d.at[pl.ds(base_chunk + sc_i * CPS, CPS)], rowv)
    pltpu.sync_copy(val2d.at[pl.ds(base_chunk + sc_i * CPS, CPS)], valv)

    fire_g(0, 0, True)
    fire_g(1, 1, True)

    def quad(q, __):
      for u in range(NBUF):
        j = q * NBUF + u
        nb = (u + 2) % NBUF
        wait_s(nb, guard=j >= 2)
        fire_g(j + 2, nb, j + 2 <= CPS - 1)
        wait_g(u)
        _scale_rows(rows[u], valv, j)
        fire_s(j, u)
      return __
    lax.fori_loop(0, CPS // NBUF, quad, None)

    wait_s(2)
    wait_s(3)
    return _
  lax.fori_loop(0, SUPER, super_step, None)


def _fused_body(zeros_hbm, t0, t1, col2d, row2d, val2d, uix, iix,
                l10, l11, l20, l21, l30, l31, usum0, usum1, isum0, isum1,
                acc, colv, rowv, valv, r0, r1, r2, r3, idxv,
                g0, g1, g2, g3, s0, s1, s2, s3):
  rows = (r0, r1, r2, r3)
  gsems = (g0, g1, g2, g3)
  ssems = (s0, s1, s2, s3)
  c = lax.axis_index("c")
  s = lax.axis_index("s")
  zsl = pl.ds(s * ROWS_PER_TILE, ROWS_PER_TILE)

  pltpu.sync_copy(zeros_hbm.at[zsl], acc.at[zsl])
  plsc.subcore_barrier()

  pairs = ((t0, t1), (l10, l11), (l20, l21), (l30, l31))
  for layer in range(LAYERS):
    cur0, cur1 = pairs[layer]
    nxt0, nxt1 = pairs[layer + 1]
    _edge_pass(c, s, cur0, cur1, col2d, row2d, val2d,
               acc, colv, rowv, valv, rows, gsems, ssems)
    plsc.subcore_barrier()
    @pl.when(c == 0)
    def _w0(nxt0=nxt0):
      pltpu.sync_copy(acc.at[zsl], nxt0.at[zsl])
    @pl.when(c == 1)
    def _w1(nxt1=nxt1):
      pltpu.sync_copy(acc.at[zsl], nxt1.at[zsl])
    if layer != LAYERS - 1:
      pltpu.sync_copy(zeros_hbm.at[zsl], acc.at[zsl])
    plsc.subcore_barrier()

  gbuf, sbuf = rows[0], rows[1]
  sem = gsems[0]

  def one_batch(idx3d, out0, out1):
    pltpu.sync_copy(idx3d.at[s], idxv)
    for j in range(BCH):
      for ti, (tb0, tb1) in enumerate(pairs):
        @pl.when(c == 0)
        def _g0(tb0=tb0, j=j):
          pltpu.async_copy(tb0.at[idxv.at[j]], gbuf, sem).wait()
        @pl.when(c == 1)
        def _g1(tb1=tb1, j=j):
          pltpu.async_copy(tb1.at[idxv.at[j]], gbuf, sem).wait()

        def body(r, _, first=(ti == 0)):
          for d2 in range(HALF // LANES):
            dsl = pl.ds(d2 * LANES, LANES)
            if first:
              sbuf[r, dsl] = gbuf[r, dsl]
            else:
              sbuf[r, dsl] = sbuf[r, dsl] + gbuf[r, dsl]
          return _
        lax.fori_loop(0, CK, body, None)
      osl = pl.ds(s * BPT + j * CK, CK)
      @pl.when(c == 0)
      def _o0(osl=osl):
        pltpu.sync_copy(sbuf, out0.at[osl])
      @pl.when(c == 1)
      def _o1(osl=osl):
        pltpu.sync_copy(sbuf, out1.at[osl])

  one_batch(uix, usum0, usum1)
  one_batch(iix, isum0, isum1)


def _make_fused_kernel():
  nt = jax.ShapeDtypeStruct((N_PAD, HALF), jnp.float32)
  bt = jax.ShapeDtypeStruct((BATCH, HALF), jnp.float32)
  return pl.kernel(
      _fused_body,
      out_type=(nt, nt, nt, nt, nt, nt, bt, bt, bt, bt),
      mesh=plsc.VectorSubcoreMesh(
          core_axis_name="c", subcore_axis_name="s",
          num_cores=NC, num_subcores=NS),
      compiler_params=pltpu.CompilerParams(use_tc_tiling_on_sc=False),
      scratch_types=(
          [pltpu.VMEM_SHARED((N_PAD, HALF), jnp.float32)]
          + [pltpu.VMEM((CPS, CK), jnp.int32)] * 2
          + [pltpu.VMEM((CPS, CK), jnp.float32)]
          + [pltpu.VMEM((CK, HALF), jnp.float32)] * NBUF
          + [pltpu.VMEM((BCH, CK), jnp.int32)]
          + [pltpu.SemaphoreType.DMA] * (2 * NBUF)
      ),
  )


def _forward_body(us_ref, is_ref, xij_ref, xt_ref, out_ref):
  ue = us_ref[...] * 0.25
  ie = is_ref[...] * 0.25
  xe = xt_ref[...] * (xij_ref[...] - 0.3).reshape(-1, 1)
  u = jnp.concatenate([ue, xe], axis=1)
  v = jnp.concatenate([ie, xe], axis=1)
  u = jax.nn.sigmoid(u)
  v = v - jnp.max(v, axis=1, keepdims=True)
  ev = jnp.exp(v)
  v = ev / jnp.sum(ev, axis=1, keepdims=True)
  out_ref[...] = jnp.sum(u * v, axis=1, keepdims=True)


def kernel(users, items, xij, edge_index, edge_vals, user_table, item_table,
           xij_table):
  all_emb = jnp.concatenate([user_table, item_table], axis=0)
  all_emb = jnp.pad(all_emb, ((0, N_PAD - N_NODES), (0, 0)))
  cur0 = all_emb[:, :HALF]
  cur1 = all_emb[:, HALF:]

  pad = E_PAD - N_EDGES
  row = jnp.pad(edge_index[0], (0, pad)).reshape(E_PAD // CK, CK)
  col = jnp.pad(edge_index[1], (0, pad)).reshape(E_PAD // CK, CK)
  val = jnp.pad(edge_vals, (0, pad)).reshape(E_PAD // CK, CK)

  zeros = jnp.zeros((N_PAD, HALF), jnp.float32)
  uix = users.reshape(NS, BCH, CK)
  iix = (items + N_USERS).reshape(NS, BCH, CK)

  fused = _make_fused_kernel()
  outs = fused(zeros, cur0, cur1, col, row, val, uix, iix)
  usum0, usum1, isum0, isum1 = outs[6:]

  usum = jnp.concatenate([usum0, usum1], axis=1)
  isum = jnp.concatenate([isum0, isum1], axis=1)
  xt = jnp.tile(xij_table, (BATCH, 1))

  gamma = pl.pallas_call(
      _forward_body,
      out_shape=jax.ShapeDtypeStruct((BATCH, 1), jnp.float32),
  )(usum, isum, xij, xt)
  return gamma.reshape(BATCH)
